# baseline (speedup 1.0000x reference)
import functools
import math

import numpy as np

import jax
import jax.numpy as jnp
from jax import lax
from jax.experimental import pallas as pl
from jax.experimental.pallas import tpu as pltpu


_HALF_LOG_2PI = 0.5 * math.log(2.0 * math.pi)
_VMEM_LIMIT = 48 * 2**20

_THIRD = np.float32(1.0 / 3.0)
_SQRT2 = np.float32(np.sqrt(2.0))
_SQUEEZE = np.float32(0.0331)
_ULO = np.nextafter(np.float32(-1.0), np.float32(0.0), dtype=np.float32)
_USPAN = np.float32(np.float32(1.0) - _ULO)
_TFC = 0x1BD11BDA


def _softplus(x):
    return jnp.where(x > 20.0, x, jnp.log1p(jnp.exp(jnp.minimum(x, 20.0))))


def _lgamma_digamma(x):
    direct = x > 8.0
    p = x * (x + 1.0) * (x + 2.0) * (x + 3.0) * (x + 4.0) * (x + 5.0)
    psum = ((((6.0 * x + 75.0) * x + 340.0) * x + 675.0) * x + 548.0) * x + 120.0
    y = jnp.where(direct, x, x + 6.0)
    r = 1.0 / y
    ly = jnp.log(y)
    r2 = r * r
    lg_corr = r * (1.0 / 12.0 - r2 * (1.0 / 360.0 - r2 * (1.0 / 1260.0 - r2 * (1.0 / 1680.0))))
    lg = (y - 0.5) * ly - y + _HALF_LOG_2PI + lg_corr
    dg = ly - 0.5 * r - r2 * (1.0 / 12.0 - r2 * (1.0 / 120.0 - r2 * (1.0 / 252.0)))
    lg = lg - jnp.where(direct, 0.0, jnp.log(p))
    dg = dg - jnp.where(direct, 0.0, psum / p)
    return lg, dg


def _kl_gamma(p_conc, p_rate, q_conc, q_rate, q_const):
    lg, dg = _lgamma_digamma(p_conc)
    return (q_conc * jnp.log(p_rate) + q_const - lg
            + (p_conc - q_conc) * dg
            + (q_rate - p_rate) * (p_conc / p_rate))


def _tf2(k1, k2, counts2):
    ks0 = k1
    ks1 = k2
    ks2 = k1 ^ k2 ^ np.uint32(_TFC)

    def rot(x, r):
        return (x << np.uint32(r)) | (x >> np.uint32(32 - r))

    def four(x0, x1, rs):
        for r in rs:
            x0 = x0 + x1
            x1 = rot(x1, r)
            x1 = x0 ^ x1
        return x0, x1

    r0 = (13, 15, 26, 6)
    r1 = (17, 29, 16, 24)
    outs = []
    for c2 in counts2:
        if isinstance(c2, int):
            c2 = np.uint32(c2)
        x0 = ks0 + jnp.zeros_like(c2 + ks1)
        x1 = c2 + ks1
        x0, x1 = four(x0, x1, r0)
        x0 = x0 + ks1
        x1 = x1 + ks2 + np.uint32(1)
        x0, x1 = four(x0, x1, r1)
        x0 = x0 + ks2
        x1 = x1 + ks0 + np.uint32(2)
        x0, x1 = four(x0, x1, r0)
        x0 = x0 + ks0
        x1 = x1 + ks1 + np.uint32(3)
        x0, x1 = four(x0, x1, r1)
        x0 = x0 + ks1
        x1 = x1 + ks2 + np.uint32(4)
        x0, x1 = four(x0, x1, r0)
        x0 = x0 + ks2
        x1 = x1 + ks0 + np.uint32(5)
        outs.append((x0, x1))
    return outs


def _unit01(bits):
    fb = (bits >> np.uint32(9)) | np.uint32(0x3F800000)
    return lax.bitcast_convert_type(fb, jnp.float32) - np.float32(1.0)


def _unit_pm1(bits):
    u = _unit01(bits) * _USPAN + _ULO
    return jnp.maximum(_ULO, u)


def _encoder_body(x_ref, w1_ref, w2_ref, b1_ref, b2_ref, qc_ref, qr_ref,
                  qk_ref, loc_ref, scale_ref, sc_ref):
    latent = loc_ref.shape[-1]
    x = x_ref[...].astype(jnp.bfloat16)
    h = jnp.dot(x, w1_ref[...], preferred_element_type=jnp.float32) + b1_ref[...]
    h = jnp.maximum(h, 0.0).astype(jnp.bfloat16)
    y = jnp.dot(h, w2_ref[...], preferred_element_type=jnp.float32) + b2_ref[...]

    loc = jnp.maximum(_softplus(y[:, :latent]), 1e-30)
    scale = jnp.maximum(_softplus(y[:, latent:]), 1e-30)
    loc_ref[...] = loc
    scale_ref[...] = scale

    kl = _kl_gamma(loc, scale, qc_ref[...], qr_ref[...], qk_ref[...])
    sc_ref[...] = jnp.sum(kl, axis=1, keepdims=True) * (1.0 / latent)


def _encode(x, w1, b1, w2, b2, q_conc, q_rate, q_const, tb):
    batch, data = x.shape
    hidden = w1.shape[1]
    latent = q_conc.shape[1]
    rep = lambda i: (0, 0)
    row = lambda i: (i, 0)
    return pl.pallas_call(
        _encoder_body,
        out_shape=(jax.ShapeDtypeStruct((batch, latent), jnp.float32),
                   jax.ShapeDtypeStruct((batch, latent), jnp.float32),
                   jax.ShapeDtypeStruct((batch, 1), jnp.float32)),
        grid=(batch // tb,),
        in_specs=[
            pl.BlockSpec((tb, data), row),
            pl.BlockSpec((data, hidden), rep),
            pl.BlockSpec((hidden, 2 * latent), rep),
            pl.BlockSpec((1, hidden), rep),
            pl.BlockSpec((1, 2 * latent), rep),
            pl.BlockSpec((1, latent), rep),
            pl.BlockSpec((1, latent), rep),
            pl.BlockSpec((1, latent), rep),
        ],
        out_specs=(pl.BlockSpec((tb, latent), row),
                   pl.BlockSpec((tb, latent), row),
                   pl.BlockSpec((tb, 1), row)),
        compiler_params=pltpu.CompilerParams(
            dimension_semantics=("parallel",),
            vmem_limit_bytes=_VMEM_LIMIT),
    )(x, w1.astype(jnp.bfloat16), w2.astype(jnp.bfloat16), b1, b2,
      q_conc, q_rate, q_const)


def _sampler_body(key_ref, loc_ref, scale_ref, qc_ref, qr_ref, qk_ref,
                  z_ref, zs_ref, *, batch):
    tb, latent = loc_ref.shape
    n = z_ref.shape[0]
    step = pl.program_id(0)

    mk1 = key_ref[0]
    mk2 = key_ref[1]

    alpha = loc_ref[...]
    scale = scale_ref[...]
    ge1 = alpha >= 1.0
    alpha_b = jnp.where(ge1, alpha, alpha + 1.0)
    d = alpha_b - _THIRD
    c = _THIRD / jnp.sqrt(d)
    inv_alpha = 1.0 / alpha

    rowi = lax.broadcasted_iota(jnp.int32, (tb, latent), 0)
    coli = lax.broadcasted_iota(jnp.int32, (tb, latent), 1)
    idx0 = (coli * batch + rowi).astype(jnp.uint32) + (step * tb).astype(jnp.uint32)

    def accept_test(U, x, v, dd):
        X = x * x
        V = (v * v) * v
        rej = ((U >= 1.0 - _SQUEEZE * (X * X))
               & (jnp.log(U) >= 0.5 * X + dd * ((1.0 - V) + jnp.log(V))))
        return V, rej

    def draw_slab(s, acc):
        idx = idx0 + (s * (latent * batch)).astype(jnp.uint32)
        ((i1, i2),) = _tf2(mk1, mk2, (idx,))
        (ka1, ka2), (bk1, bk2) = _tf2(i1, i2, (0, 1))

        (xk1, xk2), (uk1, uk2) = _tf2(ka1, ka2, (1, 2))
        (sb1, sb2), (cn1, cn2) = _tf2(xk1, xk2, (1, 0))
        ((xb1, xb2),) = _tf2(sb1, sb2, (0,))
        x = _SQRT2 * lax.erf_inv(_unit_pm1(xb1 ^ xb2))
        v = 1.0 + c * x
        ((ub1, ub2),) = _tf2(uk1, uk2, (0,))
        U = _unit01(ub1 ^ ub2)
        V, rej = accept_test(U, x, v, d)
        inner_ok = v > 0.0
        done = inner_ok & ~rej
        Vacc = jnp.where(done, V, 1.0)
        need_outer = inner_ok & rej

        def cl_cond(st):
            return jnp.any(st[7] == 0)

        def cl_body(st):
            ko1, ko2, cc1, cc2, Ucur, Vout, no_i, dn_i = st
            no = no_i != 0
            dn = dn_i != 0
            ((a1, a2),) = _tf2(ko1, ko2, (0,))
            (nx1, nx2), (nu1, nu2) = _tf2(a1, a2, (1, 2))
            ((nb1, nb2),) = _tf2(nu1, nu2, (0,))
            nU = _unit01(nb1 ^ nb2)
            ko1 = jnp.where(no, a1, ko1)
            ko2 = jnp.where(no, a2, ko2)
            cc1 = jnp.where(no, nx1, cc1)
            cc2 = jnp.where(no, nx2, cc2)
            Ucur = jnp.where(no, nU, Ucur)
            (s1, s2), (c1n, c2n) = _tf2(cc1, cc2, (1, 0))
            ((b1_, b2_),) = _tf2(s1, s2, (0,))
            xx = _SQRT2 * lax.erf_inv(_unit_pm1(b1_ ^ b2_))
            vv = 1.0 + c * xx
            VV, rj = accept_test(Ucur, xx, vv, d)
            iok = vv > 0.0
            act = ~dn
            newly = act & iok & ~rj
            Vout = jnp.where(newly, VV, Vout)
            dn = dn | newly
            no = act & iok & rj
            redraw = act & ~iok
            cc1 = jnp.where(redraw, c1n, cc1)
            cc2 = jnp.where(redraw, c2n, cc2)
            return (ko1, ko2, cc1, cc2, Ucur, Vout,
                    no.astype(jnp.int32), dn.astype(jnp.int32))

        st = lax.while_loop(
            cl_cond, cl_body,
            (ka1, ka2, cn1, cn2, U, Vacc,
             need_outer.astype(jnp.int32), done.astype(jnp.int32)))
        Vfin = st[5]

        ((bb1, bb2),) = _tf2(bk1, bk2, (0,))
        usamp = 1.0 - _unit01(bb1 ^ bb2)
        boost = jnp.where(ge1, np.float32(1.0),
                          jnp.exp(jnp.log(usamp) * inv_alpha))
        z = (d * Vfin) * boost / scale
        z_ref[pl.ds(s, 1)] = z[None]
        return acc + z

    ssum = lax.fori_loop(0, n, draw_slab,
                         jnp.zeros((tb, latent), jnp.float32))
    mz = ssum * (1.0 / n)

    def var_step(s, a):
        dv = z_ref[pl.ds(s, 1)][0] - mz
        return a + dv * dv

    sv = lax.fori_loop(0, n, var_step, jnp.zeros((tb, latent), jnp.float32))
    ms = jnp.maximum(sv * (1.0 / (n - 1)), 1e-30)
    beta = mz / ms
    a2 = mz * beta
    kl = _kl_gamma(a2, beta, qc_ref[...], qr_ref[...], qk_ref[...])
    zs_ref[...] = jnp.sum(kl, axis=1, keepdims=True) * (1.0 / latent)


def _sample(sample_key, loc, scale, q_conc, q_rate, q_const, n_samples, tb):
    batch, latent = loc.shape
    rep = lambda i: (0, 0)
    row = lambda i: (i, 0)
    return pl.pallas_call(
        functools.partial(_sampler_body, batch=batch),
        out_shape=(jax.ShapeDtypeStruct((n_samples, batch, latent), jnp.float32),
                   jax.ShapeDtypeStruct((batch, 1), jnp.float32)),
        grid=(batch // tb,),
        in_specs=[
            pl.BlockSpec(memory_space=pltpu.SMEM),
            pl.BlockSpec((tb, latent), row),
            pl.BlockSpec((tb, latent), row),
            pl.BlockSpec((1, latent), rep),
            pl.BlockSpec((1, latent), rep),
            pl.BlockSpec((1, latent), rep),
        ],
        out_specs=(pl.BlockSpec((n_samples, tb, latent), lambda i: (0, i, 0)),
                   pl.BlockSpec((tb, 1), row)),
        compiler_params=pltpu.CompilerParams(
            dimension_semantics=("parallel",),
            vmem_limit_bytes=_VMEM_LIMIT),
    )(sample_key.astype(jnp.uint32), loc, scale, q_conc, q_rate, q_const)


def kernel(x, w1, b1, w2, b2, target_conc, target_rate, sample_key):
    batch = x.shape[0]
    n_samples = 16

    q_const = jax.lax.lgamma(target_conc) - target_conc * jnp.log(target_rate)

    loc, scale, scores = _encode(x, w1, b1, w2, b2, target_conc, target_rate,
                                 q_const, tb=512)
    zrnd, z_score = _sample(sample_key, loc, scale, target_conc, target_rate,
                            q_const, n_samples, tb=32)
    return zrnd, scores, z_score


# speedup vs baseline: 1.1779x; 1.1779x over previous
import functools
import math

import numpy as np

import jax
import jax.numpy as jnp
from jax import lax
from jax.experimental import pallas as pl
from jax.experimental.pallas import tpu as pltpu


_HALF_LOG_2PI = 0.5 * math.log(2.0 * math.pi)
_VMEM_LIMIT = 48 * 2**20

_THIRD = np.float32(1.0 / 3.0)
_SQRT2 = np.float32(np.sqrt(2.0))
_SQUEEZE = np.float32(0.0331)
_ULO = np.nextafter(np.float32(-1.0), np.float32(0.0), dtype=np.float32)
_USPAN = np.float32(np.float32(1.0) - _ULO)
_TFC = 0x1BD11BDA


def _softplus(x):
    return jnp.where(x > 20.0, x, jnp.log1p(jnp.exp(jnp.minimum(x, 20.0))))


def _lgamma_digamma(x):
    direct = x > 8.0
    p = x * (x + 1.0) * (x + 2.0) * (x + 3.0) * (x + 4.0) * (x + 5.0)
    psum = ((((6.0 * x + 75.0) * x + 340.0) * x + 675.0) * x + 548.0) * x + 120.0
    y = jnp.where(direct, x, x + 6.0)
    r = 1.0 / y
    ly = jnp.log(y)
    r2 = r * r
    lg_corr = r * (1.0 / 12.0 - r2 * (1.0 / 360.0 - r2 * (1.0 / 1260.0 - r2 * (1.0 / 1680.0))))
    lg = (y - 0.5) * ly - y + _HALF_LOG_2PI + lg_corr
    dg = ly - 0.5 * r - r2 * (1.0 / 12.0 - r2 * (1.0 / 120.0 - r2 * (1.0 / 252.0)))
    lg = lg - jnp.where(direct, 0.0, jnp.log(p))
    dg = dg - jnp.where(direct, 0.0, psum / p)
    return lg, dg


def _kl_gamma(p_conc, p_rate, q_conc, q_rate, q_const):
    lg, dg = _lgamma_digamma(p_conc)
    return (q_conc * jnp.log(p_rate) + q_const - lg
            + (p_conc - q_conc) * dg
            + (q_rate - p_rate) * (p_conc / p_rate))


def _tf2(k1, k2, counts2):
    ks0 = k1
    ks1 = k2
    ks2 = k1 ^ k2 ^ np.uint32(_TFC)

    def rot(x, r):
        return (x << np.uint32(r)) | (x >> np.uint32(32 - r))

    def four(x0, x1, rs):
        for r in rs:
            x0 = x0 + x1
            x1 = rot(x1, r)
            x1 = x0 ^ x1
        return x0, x1

    r0 = (13, 15, 26, 6)
    r1 = (17, 29, 16, 24)
    outs = []
    for c2 in counts2:
        if isinstance(c2, int):
            c2 = np.uint32(c2)
        x0 = ks0 + jnp.zeros_like(c2 + ks1)
        x1 = c2 + ks1
        x0, x1 = four(x0, x1, r0)
        x0 = x0 + ks1
        x1 = x1 + ks2 + np.uint32(1)
        x0, x1 = four(x0, x1, r1)
        x0 = x0 + ks2
        x1 = x1 + ks0 + np.uint32(2)
        x0, x1 = four(x0, x1, r0)
        x0 = x0 + ks0
        x1 = x1 + ks1 + np.uint32(3)
        x0, x1 = four(x0, x1, r1)
        x0 = x0 + ks1
        x1 = x1 + ks2 + np.uint32(4)
        x0, x1 = four(x0, x1, r0)
        x0 = x0 + ks2
        x1 = x1 + ks0 + np.uint32(5)
        outs.append((x0, x1))
    return outs


def _unit01(bits):
    fb = (bits >> np.uint32(9)) | np.uint32(0x3F800000)
    return lax.bitcast_convert_type(fb, jnp.float32) - np.float32(1.0)


def _unit_pm1(bits):
    u = _unit01(bits) * _USPAN + _ULO
    return jnp.maximum(_ULO, u)


def _encoder_body(x_ref, w1_ref, w2_ref, b1_ref, b2_ref, qc_ref, qr_ref,
                  qk_ref, loc_ref, scale_ref, sc_ref):
    latent = loc_ref.shape[-1]
    x = x_ref[...].astype(jnp.bfloat16)
    h = jnp.dot(x, w1_ref[...], preferred_element_type=jnp.float32) + b1_ref[...]
    h = jnp.maximum(h, 0.0).astype(jnp.bfloat16)
    y = jnp.dot(h, w2_ref[...], preferred_element_type=jnp.float32) + b2_ref[...]

    loc = jnp.maximum(_softplus(y[:, :latent]), 1e-30)
    scale = jnp.maximum(_softplus(y[:, latent:]), 1e-30)
    loc_ref[...] = loc
    scale_ref[...] = scale

    kl = _kl_gamma(loc, scale, qc_ref[...], qr_ref[...], qk_ref[...])
    sc_ref[...] = jnp.sum(kl, axis=1, keepdims=True) * (1.0 / latent)


def _encode(x, w1, b1, w2, b2, q_conc, q_rate, q_const, tb):
    batch, data = x.shape
    hidden = w1.shape[1]
    latent = q_conc.shape[1]
    rep = lambda i: (0, 0)
    row = lambda i: (i, 0)
    return pl.pallas_call(
        _encoder_body,
        out_shape=(jax.ShapeDtypeStruct((batch, latent), jnp.float32),
                   jax.ShapeDtypeStruct((batch, latent), jnp.float32),
                   jax.ShapeDtypeStruct((batch, 1), jnp.float32)),
        grid=(batch // tb,),
        in_specs=[
            pl.BlockSpec((tb, data), row),
            pl.BlockSpec((data, hidden), rep),
            pl.BlockSpec((hidden, 2 * latent), rep),
            pl.BlockSpec((1, hidden), rep),
            pl.BlockSpec((1, 2 * latent), rep),
            pl.BlockSpec((1, latent), rep),
            pl.BlockSpec((1, latent), rep),
            pl.BlockSpec((1, latent), rep),
        ],
        out_specs=(pl.BlockSpec((tb, latent), row),
                   pl.BlockSpec((tb, latent), row),
                   pl.BlockSpec((tb, 1), row)),
        compiler_params=pltpu.CompilerParams(
            dimension_semantics=("parallel",),
            vmem_limit_bytes=_VMEM_LIMIT),
    )(x, w1.astype(jnp.bfloat16), w2.astype(jnp.bfloat16), b1, b2,
      q_conc, q_rate, q_const)


def _sampler_body(key_ref, loc_ref, scale_ref, qc_ref, qr_ref, qk_ref,
                  z_ref, zs_ref, *, batch):
    tb, latent = loc_ref.shape
    n = z_ref.shape[0]
    step = pl.program_id(0)

    mk1 = key_ref[0]
    mk2 = key_ref[1]

    alpha = loc_ref[...]
    scale = scale_ref[...]
    ge1 = alpha >= 1.0
    alpha_b = jnp.where(ge1, alpha, alpha + 1.0)
    d = alpha_b - _THIRD
    c = _THIRD / jnp.sqrt(d)
    inv_alpha = 1.0 / alpha

    rowi = lax.broadcasted_iota(jnp.int32, (tb, latent), 0)
    coli = lax.broadcasted_iota(jnp.int32, (tb, latent), 1)
    idx0 = (coli * batch + rowi).astype(jnp.uint32) + (step * tb).astype(jnp.uint32)

    def accept_test(U, x, v, dd):
        X = x * x
        V = (v * v) * v
        rej = ((U >= 1.0 - _SQUEEZE * (X * X))
               & (jnp.log(U) >= 0.5 * X + dd * ((1.0 - V) + jnp.log(V))))
        return V, rej

    def draw_slab(s, acc):
        idx = idx0 + (s * (latent * batch)).astype(jnp.uint32)
        ((i1, i2),) = _tf2(mk1, mk2, (idx,))
        (ka1, ka2), (bk1, bk2) = _tf2(i1, i2, (0, 1))

        (xk1, xk2), (uk1, uk2) = _tf2(ka1, ka2, (1, 2))
        ((sb1, sb2),) = _tf2(xk1, xk2, (1,))
        ((xb1, xb2),) = _tf2(sb1, sb2, (0,))
        x = _SQRT2 * lax.erf_inv(_unit_pm1(xb1 ^ xb2))
        v = 1.0 + c * x
        ((ub1, ub2),) = _tf2(uk1, uk2, (0,))
        U = _unit01(ub1 ^ ub2)
        V, rej = accept_test(U, x, v, d)
        inner_ok = v > 0.0
        done = inner_ok & ~rej
        Vacc = jnp.where(done, V, 1.0)

        ((bb1, bb2),) = _tf2(bk1, bk2, (0,))
        usamp = 1.0 - _unit01(bb1 ^ bb2)
        boost = jnp.where(ge1, np.float32(1.0),
                          jnp.exp(jnp.log(usamp) * inv_alpha))

        fl = ((inner_ok & rej).astype(jnp.int32)
              + (~inner_ok).astype(jnp.int32) * 2
              + done.astype(jnp.int32) * 4)

        def cl_cond(st):
            return jnp.any((st[6] & 4) == 0)

        def cl_body(st):
            ko1, ko2, cc1, cc2, Ucur, Vout, flw = st
            no = (flw & 1) != 0
            rd = (flw & 2) != 0
            dn = (flw & 4) != 0
            ((a1, a2),) = _tf2(ko1, ko2, (0,))
            (nx1, nx2), (nu1, nu2) = _tf2(a1, a2, (1, 2))
            ((nb1, nb2),) = _tf2(nu1, nu2, (0,))
            ((ci1, ci2),) = _tf2(cc1, cc2, (0,))
            ko1 = jnp.where(no, a1, ko1)
            ko2 = jnp.where(no, a2, ko2)
            cc1 = jnp.where(no, nx1, jnp.where(rd, ci1, cc1))
            cc2 = jnp.where(no, nx2, jnp.where(rd, ci2, cc2))
            Ucur = jnp.where(no, _unit01(nb1 ^ nb2), Ucur)
            ((s1, s2),) = _tf2(cc1, cc2, (1,))
            ((b1_, b2_),) = _tf2(s1, s2, (0,))
            xx = _SQRT2 * lax.erf_inv(_unit_pm1(b1_ ^ b2_))
            vv = 1.0 + c * xx
            VV, rj = accept_test(Ucur, xx, vv, d)
            iok = vv > 0.0
            act = ~dn
            newly = act & iok & ~rj
            Vout = jnp.where(newly, VV, Vout)
            dn = dn | newly
            nfl = ((act & iok & rj).astype(jnp.int32)
                   + (act & ~iok).astype(jnp.int32) * 2
                   + dn.astype(jnp.int32) * 4)
            return (ko1, ko2, cc1, cc2, Ucur, Vout, nfl)

        st = cl_body((ka1, ka2, xk1, xk2, U, Vacc, fl))
        st = lax.while_loop(cl_cond, cl_body, st)
        Vfin = st[5]

        z = (d * Vfin) * boost / scale
        z_ref[pl.ds(s, 1)] = z[None]
        return acc + z

    ssum = lax.fori_loop(0, n, draw_slab,
                         jnp.zeros((tb, latent), jnp.float32))
    mz = ssum * (1.0 / n)

    def var_step(s, a):
        dv = z_ref[pl.ds(s, 1)][0] - mz
        return a + dv * dv

    sv = lax.fori_loop(0, n, var_step, jnp.zeros((tb, latent), jnp.float32))
    ms = jnp.maximum(sv * (1.0 / (n - 1)), 1e-30)
    beta = mz / ms
    a2 = mz * beta
    kl = _kl_gamma(a2, beta, qc_ref[...], qr_ref[...], qk_ref[...])
    zs_ref[...] = jnp.sum(kl, axis=1, keepdims=True) * (1.0 / latent)


def _sample(sample_key, loc, scale, q_conc, q_rate, q_const, n_samples, tb):
    batch, latent = loc.shape
    rep = lambda i: (0, 0)
    row = lambda i: (i, 0)
    return pl.pallas_call(
        functools.partial(_sampler_body, batch=batch),
        out_shape=(jax.ShapeDtypeStruct((n_samples, batch, latent), jnp.float32),
                   jax.ShapeDtypeStruct((batch, 1), jnp.float32)),
        grid=(batch // tb,),
        in_specs=[
            pl.BlockSpec(memory_space=pltpu.SMEM),
            pl.BlockSpec((tb, latent), row),
            pl.BlockSpec((tb, latent), row),
            pl.BlockSpec((1, latent), rep),
            pl.BlockSpec((1, latent), rep),
            pl.BlockSpec((1, latent), rep),
        ],
        out_specs=(pl.BlockSpec((n_samples, tb, latent), lambda i: (0, i, 0)),
                   pl.BlockSpec((tb, 1), row)),
        compiler_params=pltpu.CompilerParams(
            dimension_semantics=("parallel",),
            vmem_limit_bytes=_VMEM_LIMIT),
    )(sample_key.astype(jnp.uint32), loc, scale, q_conc, q_rate, q_const)


def kernel(x, w1, b1, w2, b2, target_conc, target_rate, sample_key):
    batch = x.shape[0]
    n_samples = 16

    q_const = jax.lax.lgamma(target_conc) - target_conc * jnp.log(target_rate)

    loc, scale, scores = _encode(x, w1, b1, w2, b2, target_conc, target_rate,
                                 q_const, tb=512)
    zrnd, z_score = _sample(sample_key, loc, scale, target_conc, target_rate,
                            q_const, n_samples, tb=64)
    return zrnd, scores, z_score
